# Initial kernel scaffold; baseline (speedup 1.0000x reference)
import jax
import jax.numpy as jnp
from jax.experimental import pallas as pl


def kernel(input_current, vb_t, A_t, th_t, gain_t, tref_t):
    raise NotImplementedError("write your pallas kernel here")


# speedup vs baseline: 39.2152x; 39.2152x over previous
import functools

import jax
import jax.numpy as jnp
from jax import lax
from jax.experimental import pallas as pl
from jax.experimental.pallas import tpu as pltpu

_DT = 0.05
_TAU_DC = 50.0
_ALPHA = _DT / max(_TAU_DC, 1e-6)

_B_BLK = 8
_T_BLK = 256


def _scan_kernel(nt, t_blk,
                 x_ref, vb_ref, a_ref, th_ref, gain_ref, rs_ref,
                 va_ref, sp_ref,
                 ema_ref, refc_ref, vac_ref, spc_ref):
    j = pl.program_id(1)
    vb = vb_ref[...]
    a = a_ref[...]
    th = th_ref[...]
    gain = gain_ref[...]
    rs = rs_ref[...]
    bblk, f = ema_ref.shape

    @pl.when(j == 0)
    def _():
        va_ref[:, 0, :] = jnp.broadcast_to(vb, (bblk, f))
        ema_ref[...] = x_ref[:, 0, :] * gain
        refc_ref[...] = jnp.zeros((bblk, f), jnp.int32)

    @pl.when(j > 0)
    def _():
        va_ref[:, 0, :] = vac_ref[...]

    @pl.when(j == nt)
    def _():
        sp_ref[:, 0, :] = spc_ref[...]

    @pl.when(j < nt)
    def _():
        def step(xt_raw, ema, refc):
            xt = xt_raw * gain
            ema = ema + _ALPHA * (xt - ema)
            va_cand = vb - a * (xt - ema)
            in_ref = refc > 0
            fired = jnp.logical_and(jnp.logical_not(in_ref),
                                    jnp.abs(va_cand - vb) >= th)
            va_next = jnp.where(jnp.logical_or(in_ref, fired), vb, va_cand)
            refc = jnp.where(in_ref, refc - 1, refc)
            refc = jnp.where(fired, rs, refc)
            return ema, refc, va_next, fired

        def body(k, carry):
            ema, refc = carry
            ema, refc, va_next, fired = step(x_ref[:, k, :], ema, refc)
            sp_ref[:, k, :] = fired
            va_ref[:, k + 1, :] = va_next
            return ema, refc

        ema, refc = lax.fori_loop(0, t_blk - 1, body,
                                  (ema_ref[...], refc_ref[...]), unroll=8)
        ema, refc, va_next, fired = step(x_ref[:, t_blk - 1, :], ema, refc)
        sp_ref[:, t_blk - 1, :] = fired
        ema_ref[...] = ema
        refc_ref[...] = refc
        vac_ref[...] = va_next
        spc_ref[...] = fired


def kernel(input_current, vb_t, A_t, th_t, gain_t, tref_t):
    x = input_current
    b, t, f = x.shape
    bblk = _B_BLK if b % _B_BLK == 0 else b
    tblk = _T_BLK if t % _T_BLK == 0 else t
    nb = b // bblk
    nt = t // tblk

    ref_steps = jnp.maximum(jnp.ceil(tref_t / max(_DT, 1e-6)), 1.0)
    ref_steps = ref_steps.astype(jnp.int32)

    def row(v):
        return v.reshape(1, f)

    param_spec = pl.BlockSpec((1, f), lambda i, j: (0, 0))
    va, sp = pl.pallas_call(
        functools.partial(_scan_kernel, nt, tblk),
        grid=(nb, nt + 1),
        in_specs=[
            pl.BlockSpec((bblk, tblk, f),
                         lambda i, j: (i, jnp.minimum(j, nt - 1), 0)),
            param_spec, param_spec, param_spec, param_spec, param_spec,
        ],
        out_specs=[
            pl.BlockSpec((bblk, tblk, f), lambda i, j: (i, j, 0)),
            pl.BlockSpec((bblk, tblk, f), lambda i, j: (i, j, 0)),
        ],
        out_shape=[
            jax.ShapeDtypeStruct((b, t + 1, f), x.dtype),
            jax.ShapeDtypeStruct((b, t + 1, f), jnp.bool_),
        ],
        scratch_shapes=[
            pltpu.VMEM((bblk, f), jnp.float32),
            pltpu.VMEM((bblk, f), jnp.int32),
            pltpu.VMEM((bblk, f), jnp.float32),
            pltpu.VMEM((bblk, f), jnp.bool_),
        ],
        compiler_params=pltpu.CompilerParams(
            dimension_semantics=("parallel", "arbitrary"),
        ),
        name="faneuron_scan",
    )(x, row(vb_t), row(A_t), row(th_t), row(gain_t), row(ref_steps))
    return va, sp


# speedup vs baseline: 48.3857x; 1.2339x over previous
import functools

import jax
import jax.numpy as jnp
from jax import lax
from jax.experimental import pallas as pl
from jax.experimental.pallas import tpu as pltpu

_DT = 0.05
_TAU_DC = 50.0
_ALPHA = _DT / max(_TAU_DC, 1e-6)

_B_BLK = 8
_T_BLK = 256


def _scan_kernel(nt, t_blk,
                 x_ref, vb_ref, a_ref, th_ref, gain_ref, rs_ref,
                 va_ref, sp_ref,
                 xs_ref, vas_ref, sps_ref,
                 ema_ref, refc_ref, vac_ref, spc_ref):
    j = pl.program_id(1)
    vb = vb_ref[...]
    a = a_ref[...]
    th = th_ref[...]
    gain = gain_ref[...]
    rs = rs_ref[...]
    bblk, f = ema_ref.shape

    @pl.when(j < nt)
    def _():
        for tb in range(0, t_blk, 8):
            xs_ref[pl.ds(tb, 8)] = jnp.swapaxes(
                x_ref[:, pl.ds(tb, 8), :], 0, 1)

        @pl.when(j == 0)
        def _():
            ema_ref[...] = xs_ref[0] * gain
            refc_ref[...] = jnp.zeros((bblk, f), jnp.int32)
            vac_ref[...] = jnp.broadcast_to(vb, (bblk, f))

        vas_ref[0] = vac_ref[...]

        def step(xt_raw, ema, refc):
            xt = xt_raw * gain
            ema = ema + _ALPHA * (xt - ema)
            va_cand = vb - a * (xt - ema)
            in_ref = refc > 0
            fired = jnp.logical_and(jnp.logical_not(in_ref),
                                    jnp.abs(va_cand - vb) >= th)
            va_next = jnp.where(jnp.logical_or(in_ref, fired), vb, va_cand)
            refc = jnp.where(in_ref, refc - 1, refc)
            refc = jnp.where(fired, rs, refc)
            return ema, refc, va_next, fired

        def body(k, carry):
            ema, refc = carry
            ema, refc, va_next, fired = step(xs_ref[k], ema, refc)
            sps_ref[k] = fired.astype(jnp.int32)
            vas_ref[k + 1] = va_next
            return ema, refc

        ema, refc = lax.fori_loop(0, t_blk - 1, body,
                                  (ema_ref[...], refc_ref[...]), unroll=8)
        ema, refc, va_next, fired = step(xs_ref[t_blk - 1], ema, refc)
        sps_ref[t_blk - 1] = fired.astype(jnp.int32)
        ema_ref[...] = ema
        refc_ref[...] = refc
        vac_ref[...] = va_next
        spc_ref[...] = fired

        for tb in range(0, t_blk, 8):
            va_ref[:, pl.ds(tb, 8), :] = jnp.swapaxes(
                vas_ref[pl.ds(tb, 8)], 0, 1)
            sp_ref[:, pl.ds(tb, 8), :] = jnp.swapaxes(
                sps_ref[pl.ds(tb, 8)], 0, 1) != 0

    @pl.when(j == nt)
    def _():
        va_ref[:, 0, :] = vac_ref[...]
        sp_ref[:, 0, :] = spc_ref[...]


def kernel(input_current, vb_t, A_t, th_t, gain_t, tref_t):
    x = input_current
    b, t, f = x.shape
    bblk = _B_BLK if b % _B_BLK == 0 else b
    tblk = _T_BLK if t % _T_BLK == 0 else t
    nb = b // bblk
    nt = t // tblk

    ref_steps = jnp.maximum(jnp.ceil(tref_t / max(_DT, 1e-6)), 1.0)
    ref_steps = ref_steps.astype(jnp.int32)

    def row(v):
        return v.reshape(1, f)

    param_spec = pl.BlockSpec((1, f), lambda i, j: (0, 0))
    va, sp = pl.pallas_call(
        functools.partial(_scan_kernel, nt, tblk),
        grid=(nb, nt + 1),
        in_specs=[
            pl.BlockSpec((bblk, tblk, f),
                         lambda i, j: (i, jnp.minimum(j, nt - 1), 0)),
            param_spec, param_spec, param_spec, param_spec, param_spec,
        ],
        out_specs=[
            pl.BlockSpec((bblk, tblk, f), lambda i, j: (i, j, 0)),
            pl.BlockSpec((bblk, tblk, f), lambda i, j: (i, j, 0)),
        ],
        out_shape=[
            jax.ShapeDtypeStruct((b, t + 1, f), x.dtype),
            jax.ShapeDtypeStruct((b, t + 1, f), jnp.bool_),
        ],
        scratch_shapes=[
            pltpu.VMEM((tblk, bblk, f), jnp.float32),
            pltpu.VMEM((tblk, bblk, f), jnp.float32),
            pltpu.VMEM((tblk, bblk, f), jnp.int32),
            pltpu.VMEM((bblk, f), jnp.float32),
            pltpu.VMEM((bblk, f), jnp.int32),
            pltpu.VMEM((bblk, f), jnp.float32),
            pltpu.VMEM((bblk, f), jnp.bool_),
        ],
        compiler_params=pltpu.CompilerParams(
            dimension_semantics=("parallel", "arbitrary"),
        ),
        name="faneuron_scan",
    )(x, row(vb_t), row(A_t), row(th_t), row(gain_t), row(ref_steps))
    return va, sp


# speedup vs baseline: 51.0616x; 1.0553x over previous
import functools

import jax
import jax.numpy as jnp
from jax import lax
from jax.experimental import pallas as pl
from jax.experimental.pallas import tpu as pltpu

_DT = 0.05
_TAU_DC = 50.0
_ALPHA = _DT / max(_TAU_DC, 1e-6)

_B_BLK = 8
_T_BLK = 256


def _scan_kernel(nt, t_blk,
                 x_ref, vb_ref, a_ref, th_ref, gain_ref, rs_ref,
                 va_ref, sp_ref,
                 xs_ref, vas_ref, sps_ref,
                 ema_ref, refc_ref, vac_ref, spc_ref):
    j = pl.program_id(1)
    vb = vb_ref[...]
    a = a_ref[...]
    th = th_ref[...]
    gain = gain_ref[...]
    rs = rs_ref[...]
    bblk, f = ema_ref.shape

    @pl.when(j < nt)
    def _():
        for tb in range(0, t_blk, 8):
            xs_ref[pl.ds(tb, 8)] = jnp.swapaxes(
                x_ref[:, pl.ds(tb, 8), :], 0, 1)

        @pl.when(j == 0)
        def _():
            ema_ref[...] = xs_ref[0] * gain
            refc_ref[...] = jnp.zeros((bblk, f), jnp.int32)
            vac_ref[...] = jnp.broadcast_to(vb, (bblk, f))

        vas_ref[0] = vac_ref[...]

        def step(xt_raw, ema, refc):
            xt = xt_raw * gain
            ema = ema + _ALPHA * (xt - ema)
            va_cand = vb - a * (xt - ema)
            in_ref = refc > 0
            fired = jnp.logical_and(jnp.logical_not(in_ref),
                                    jnp.abs(va_cand - vb) >= th)
            va_next = jnp.where(jnp.logical_or(in_ref, fired), vb, va_cand)
            refc = jnp.where(in_ref, refc - 1, refc)
            refc = jnp.where(fired, rs, refc)
            return ema, refc, va_next, fired

        def body(k, carry):
            ema, refc = carry
            ema, refc, va_next, fired = step(xs_ref[k], ema, refc)
            sps_ref[k] = fired.astype(jnp.int32)
            vas_ref[k + 1] = va_next
            return ema, refc

        ema, refc = lax.fori_loop(0, t_blk - 1, body,
                                  (ema_ref[...], refc_ref[...]), unroll=8)
        ema, refc, va_next, fired = step(xs_ref[t_blk - 1], ema, refc)
        sps_ref[t_blk - 1] = fired.astype(jnp.int32)
        ema_ref[...] = ema
        refc_ref[...] = refc
        vac_ref[...] = va_next
        spc_ref[...] = fired

        for tb in range(0, t_blk, 8):
            va_ref[:, pl.ds(tb, 8), :] = jnp.swapaxes(
                vas_ref[pl.ds(tb, 8)], 0, 1)
            sp_ref[:, pl.ds(tb, 8), :] = (jnp.swapaxes(
                sps_ref[pl.ds(tb, 8)], 0, 1) != 0).astype(jnp.int8)

    @pl.when(j == nt)
    def _():
        va_ref[:, 0, :] = vac_ref[...]
        sp_ref[:, 0, :] = spc_ref[...].astype(jnp.int8)


def kernel(input_current, vb_t, A_t, th_t, gain_t, tref_t):
    x = input_current
    b, t, f = x.shape
    bblk = _B_BLK if b % _B_BLK == 0 else b
    tblk = _T_BLK if t % _T_BLK == 0 else t
    nb = b // bblk
    nt = t // tblk

    ref_steps = jnp.maximum(jnp.ceil(tref_t / max(_DT, 1e-6)), 1.0)
    ref_steps = ref_steps.astype(jnp.int32)

    def row(v):
        return v.reshape(1, f)

    param_spec = pl.BlockSpec((1, f), lambda i, j: (0, 0))
    va, sp = pl.pallas_call(
        functools.partial(_scan_kernel, nt, tblk),
        grid=(nb, nt + 1),
        in_specs=[
            pl.BlockSpec((bblk, tblk, f),
                         lambda i, j: (i, jnp.minimum(j, nt - 1), 0)),
            param_spec, param_spec, param_spec, param_spec, param_spec,
        ],
        out_specs=[
            pl.BlockSpec((bblk, tblk, f), lambda i, j: (i, j, 0)),
            pl.BlockSpec((bblk, tblk, f), lambda i, j: (i, j, 0)),
        ],
        out_shape=[
            jax.ShapeDtypeStruct((b, t + 1, f), x.dtype),
            jax.ShapeDtypeStruct((b, t + 1, f), jnp.int8),
        ],
        scratch_shapes=[
            pltpu.VMEM((tblk, bblk, f), jnp.float32),
            pltpu.VMEM((tblk, bblk, f), jnp.float32),
            pltpu.VMEM((tblk, bblk, f), jnp.int32),
            pltpu.VMEM((bblk, f), jnp.float32),
            pltpu.VMEM((bblk, f), jnp.int32),
            pltpu.VMEM((bblk, f), jnp.float32),
            pltpu.VMEM((bblk, f), jnp.bool_),
        ],
        compiler_params=pltpu.CompilerParams(
            dimension_semantics=("parallel", "arbitrary"),
        ),
        name="faneuron_scan",
    )(x, row(vb_t), row(A_t), row(th_t), row(gain_t), row(ref_steps))
    return va, sp.astype(jnp.bool_)


# speedup vs baseline: 103.6809x; 2.0305x over previous
import functools

import jax
import jax.numpy as jnp
from jax import lax
from jax.experimental import pallas as pl
from jax.experimental.pallas import tpu as pltpu

_DT = 0.05
_TAU_DC = 50.0
_ALPHA = _DT / max(_TAU_DC, 1e-6)

_B_BLK = 8
_T_BLK = 256


def _scan_kernel(nt, t_blk,
                 x_ref, vb_ref, a_ref, th_ref, gain_ref, rs_ref,
                 va_ref, sp_ref,
                 xs_ref, ema_ref, refc_ref, vac_ref, spc_ref):
    j = pl.program_id(1)
    vb = vb_ref[...]
    a = a_ref[...]
    th = th_ref[...]
    gain = gain_ref[...]
    rs = rs_ref[...]
    bblk, f = ema_ref.shape

    @pl.when(j < nt)
    def _():
        for tb in range(0, t_blk, 8):
            xs_ref[pl.ds(tb, 8)] = jnp.swapaxes(
                x_ref[:, pl.ds(tb, 8), :], 0, 1)

        @pl.when(j == 0)
        def _():
            ema_ref[...] = xs_ref[0] * gain
            refc_ref[...] = jnp.zeros((bblk, f), jnp.int32)
            vac_ref[...] = jnp.broadcast_to(vb, (bblk, f))

        va_ref[0] = vac_ref[...]

        def step(xt_raw, ema, refc):
            xt = xt_raw * gain
            ema = ema + _ALPHA * (xt - ema)
            va_cand = vb - a * (xt - ema)
            in_ref = refc > 0
            fired = jnp.logical_and(jnp.logical_not(in_ref),
                                    jnp.abs(va_cand - vb) >= th)
            va_next = jnp.where(jnp.logical_or(in_ref, fired), vb, va_cand)
            refc = jnp.where(in_ref, refc - 1, refc)
            refc = jnp.where(fired, rs, refc)
            return ema, refc, va_next, fired

        def body(k, carry):
            ema, refc = carry
            ema, refc, va_next, fired = step(xs_ref[k], ema, refc)
            sp_ref[k] = fired.astype(jnp.int8)
            va_ref[k + 1] = va_next
            return ema, refc

        ema, refc = lax.fori_loop(0, t_blk - 1, body,
                                  (ema_ref[...], refc_ref[...]), unroll=8)
        ema, refc, va_next, fired = step(xs_ref[t_blk - 1], ema, refc)
        sp_ref[t_blk - 1] = fired.astype(jnp.int8)
        ema_ref[...] = ema
        refc_ref[...] = refc
        vac_ref[...] = va_next
        spc_ref[...] = fired

    @pl.when(j == nt)
    def _():
        va_ref[0] = vac_ref[...]
        sp_ref[0] = spc_ref[...].astype(jnp.int8)


def kernel(input_current, vb_t, A_t, th_t, gain_t, tref_t):
    x = input_current
    b, t, f = x.shape
    bblk = _B_BLK if b % _B_BLK == 0 else b
    tblk = _T_BLK if t % _T_BLK == 0 else t
    nb = b // bblk
    nt = t // tblk

    ref_steps = jnp.maximum(jnp.ceil(tref_t / max(_DT, 1e-6)), 1.0)
    ref_steps = ref_steps.astype(jnp.int32)

    def row(v):
        return v.reshape(1, f)

    param_spec = pl.BlockSpec((1, f), lambda i, j: (0, 0))
    va_tr, sp_tr = pl.pallas_call(
        functools.partial(_scan_kernel, nt, tblk),
        grid=(nb, nt + 1),
        in_specs=[
            pl.BlockSpec((bblk, tblk, f),
                         lambda i, j: (i, jnp.minimum(j, nt - 1), 0)),
            param_spec, param_spec, param_spec, param_spec, param_spec,
        ],
        out_specs=[
            pl.BlockSpec((tblk, bblk, f), lambda i, j: (j, i, 0)),
            pl.BlockSpec((tblk, bblk, f), lambda i, j: (j, i, 0)),
        ],
        out_shape=[
            jax.ShapeDtypeStruct((t + 1, b, f), x.dtype),
            jax.ShapeDtypeStruct((t + 1, b, f), jnp.int8),
        ],
        scratch_shapes=[
            pltpu.VMEM((tblk, bblk, f), jnp.float32),
            pltpu.VMEM((bblk, f), jnp.float32),
            pltpu.VMEM((bblk, f), jnp.int32),
            pltpu.VMEM((bblk, f), jnp.float32),
            pltpu.VMEM((bblk, f), jnp.bool_),
        ],
        compiler_params=pltpu.CompilerParams(
            dimension_semantics=("parallel", "arbitrary"),
        ),
        name="faneuron_scan",
    )(x, row(vb_t), row(A_t), row(th_t), row(gain_t), row(ref_steps))
    va = jnp.swapaxes(va_tr, 0, 1)
    sp = jnp.swapaxes(sp_tr, 0, 1).astype(jnp.bool_)
    return va, sp
